# baseline (speedup 1.0000x reference)
import functools

import jax
import jax.numpy as jnp
from jax import lax
from jax.experimental import pallas as pl
from jax.experimental.pallas import tpu as pltpu
from jax.experimental.pallas import tpu_sc as plsc

_NUM_CORES = 2
_NUM_SUBCORES = 16
_NUM_WORKERS = _NUM_CORES * _NUM_SUBCORES

_GATHER_W = 128
_K = 4
_CHUNK = _GATHER_W * _K


def _gather_kernel_body(n_per_w, n_chunks, table_hbm, idx_hbm, out_hbm,
                        idx_v, rows_v, sem):
  wid = lax.axis_index("s") * _NUM_CORES + lax.axis_index("c")
  base = wid * n_per_w
  pltpu.sync_copy(idx_hbm.at[pl.ds(base, n_per_w)], idx_v)

  def chunk_body(ci, _):
    off = ci * _CHUNK
    copies = []
    for j in range(_K):
      src = table_hbm.at[idx_v.at[pl.ds(off + j * _GATHER_W, _GATHER_W)]]
      dst = rows_v.at[pl.ds(j * _GATHER_W, _GATHER_W)]
      copies.append(pltpu.async_copy(src, dst, sem))
    for c in copies:
      c.wait()
    pltpu.sync_copy(rows_v, out_hbm.at[pl.ds(base + off, _CHUNK)])
    return 0

  lax.fori_loop(0, n_chunks, chunk_body, 0)


def kernel(token_ids, W):
  B, S = token_ids.shape
  V, D = W.shape
  n = B * S
  assert n % (_NUM_WORKERS * _CHUNK) == 0
  n_per_w = n // _NUM_WORKERS
  n_chunks = n_per_w // _CHUNK

  idx = token_ids.reshape(n).astype(jnp.int32)

  mesh = plsc.VectorSubcoreMesh(core_axis_name="c", subcore_axis_name="s")
  gather = pl.kernel(
      functools.partial(_gather_kernel_body, n_per_w, n_chunks),
      out_type=jax.ShapeDtypeStruct((n, D), jnp.float32),
      mesh=mesh,
      scratch_types=[
          pltpu.VMEM((n_per_w,), jnp.int32),
          pltpu.VMEM((_CHUNK, D), jnp.float32),
          pltpu.SemaphoreType.DMA,
      ],
      compiler_params=pltpu.CompilerParams(use_tc_tiling_on_sc=False),
  )
  out = gather(W, idx)
  return out.reshape(B, S, D)
